# baseline (speedup 1.0000x reference)
import functools

import jax
import jax.numpy as jnp
from jax import lax
from jax.experimental import pallas as pl
from jax.experimental.pallas import tpu as pltpu
from jax.experimental.pallas import tpu_sc as plsc

B, N1, N2 = 8, 4096, 1024
C1, C2 = 128, 256
NBLK = 256
NBLK_B = 2048
NBLK_S = 1024

NW = 32
RPW = (B * N1) // NW
CH = 16
NCH = RPW // CH


def _pass_a_body(xyz1_ref, xyz2t_ref, feat1_ref, feat2_ref, w0a_ref, w0b_ref,
                 b0_ref, idx_ref, wn_ref, y0a_ref, table_ref):
    b = pl.program_id(0)
    i = pl.program_id(1)

    @pl.when(i == 0)
    def _():
        table_ref[0] = jax.lax.dot_general(
            feat2_ref[0], w0b_ref[...], (((0,), (1,)), ((), ())),
            preferred_element_type=jnp.float32)

    x1 = xyz1_ref[0]
    x2 = xyz2t_ref[0]
    dx = x1[:, 0:1] - x2[0:1, :]
    dy = x1[:, 1:2] - x2[1:2, :]
    dz = x1[:, 2:3] - x2[2:3, :]
    d = (dx * dx + dy * dy) + dz * dz

    iota_i = jax.lax.broadcasted_iota(jnp.int32, (NBLK, N2), 1)
    db = jax.lax.bitcast_convert_type(d, jnp.int32)
    key = (db & jnp.int32(-1024)) | iota_i
    bigi = jnp.int32(0x7FFFFFFF)
    k1 = jnp.min(key, axis=1, keepdims=True)
    key2 = jnp.where(key == k1, bigi, key)
    k2 = jnp.min(key2, axis=1, keepdims=True)
    key3 = jnp.where(key2 == k2, bigi, key2)
    k3 = jnp.min(key3, axis=1, keepdims=True)
    i1 = k1 & 1023
    i2 = k2 & 1023
    i3 = k3 & 1023
    m1 = jax.lax.bitcast_convert_type(k1 & jnp.int32(-1024), jnp.float32)
    m2 = jax.lax.bitcast_convert_type(k2 & jnp.int32(-1024), jnp.float32)
    m3 = jax.lax.bitcast_convert_type(k3 & jnp.int32(-1024), jnp.float32)
    w1 = 1.0 / (m1 + 1e-8)
    w2 = 1.0 / (m2 + 1e-8)
    w3 = 1.0 / (m3 + 1e-8)
    wsum = w1 + w2 + w3

    idx_ref[0] = jnp.concatenate([i1, i2, i3], axis=1) + b * N2
    ones16 = jnp.ones((1, 16), jnp.float32)
    wn_ref[0] = jnp.concatenate(
        [(w1 / wsum) * ones16, (w2 / wsum) * ones16, (w3 / wsum) * ones16],
        axis=1)

    y0a_ref[0] = jax.lax.dot_general(
        feat1_ref[0], w0a_ref[...], (((0,), (1,)), ((), ())),
        preferred_element_type=jnp.float32) + b0_ref[...]


def _sc_body(table_hbm, idx_hbm, w_hbm, y0_hbm,
             idx_all, w_v, rows_v, out_v,
             gsem0, gsem1, osem0, osem1):
    wid = lax.axis_index("s") * 2 + lax.axis_index("c")
    base = wid * RPW
    CH3 = CH * 3
    gsems = (gsem0, gsem1)
    osems = (osem0, osem1)

    pltpu.sync_copy(idx_hbm.at[pl.ds(base * 3, RPW * 3)], idx_all)

    def start(t, s):
        r0 = base + t * CH
        pltpu.async_copy(table_hbm.at[idx_all.at[pl.ds(t * CH3, CH3)]],
                         rows_v.at[s], gsems[s])
        pltpu.async_copy(w_hbm.at[pl.ds(r0 * 3, CH3)], w_v.at[s], gsems[s])

    def wait_in(t, s):
        pltpu.make_async_copy(table_hbm.at[idx_all.at[pl.ds(t * CH3, CH3)]],
                              rows_v.at[s], gsems[s]).wait()
        pltpu.make_async_copy(w_hbm.at[pl.ds(0, CH3)], w_v.at[s],
                              gsems[s]).wait()

    def wait_out(os_):
        pltpu.make_async_copy(out_v.at[os_], y0_hbm.at[pl.ds(0, CH)],
                              osems[os_]).wait()

    def compute(t, s, os_):
        r0 = base + t * CH
        for q in range(CH):
            w0 = w_v[s, 3 * q, :]
            w1 = w_v[s, 3 * q + 1, :]
            w2 = w_v[s, 3 * q + 2, :]
            for c in range(C2 // 16):
                sl = pl.ds(c * 16, 16)
                acc = (w0 * rows_v[s, 3 * q, sl]
                       + w1 * rows_v[s, 3 * q + 1, sl]
                       + w2 * rows_v[s, 3 * q + 2, sl])
                out_v[os_, q, sl] = acc
        pltpu.async_copy(out_v.at[os_], y0_hbm.at[pl.ds(r0, CH)], osems[os_])

    start(0, 0)
    start(1, 1)

    def body(tt, carry):
        t0 = tt * 2
        for s in range(2):
            t = t0 + s
            wait_in(t, s)

            @pl.when(t >= 2)
            def _():
                wait_out(s)

            compute(t, s, s)

            @pl.when(t + 2 < NCH)
            def _():
                start(t + 2, s)

        return carry

    lax.fori_loop(0, NCH // 2, body, 0)
    wait_out(0)
    wait_out(1)


def _pass_s_body(interp_ref, y0a_ref, y0_ref, s0_ref):
    b = pl.program_id(0)
    i = pl.program_id(1)
    y0 = interp_ref[0] + y0a_ref[0]
    y0_ref[0] = y0
    part = jnp.concatenate(
        [jnp.sum(y0, axis=0, keepdims=True),
         jnp.sum(y0 * y0, axis=0, keepdims=True)], axis=0)

    @pl.when((b == 0) & (i == 0))
    def _():
        s0_ref[...] = part

    @pl.when(~((b == 0) & (i == 0)))
    def _():
        s0_ref[...] = s0_ref[...] + part


def _pass_b_body(y0_ref, a0_ref, c0_ref, w1_ref, b1_ref, y1_ref, s1_ref):
    b = pl.program_id(0)
    i = pl.program_id(1)
    h0 = jnp.maximum(y0_ref[0] * a0_ref[...] + c0_ref[...], 0.0)
    y1 = jax.lax.dot_general(
        w1_ref[...], h0, (((1,), (1,)), ((), ())),
        preferred_element_type=jnp.float32) + b1_ref[...]
    y1_ref[0] = y1

    part = jnp.concatenate(
        [jnp.sum(y1, axis=1, keepdims=True),
         jnp.sum(y1 * y1, axis=1, keepdims=True)], axis=1)

    @pl.when((b == 0) & (i == 0))
    def _():
        s1_ref[...] = part

    @pl.when(~((b == 0) & (i == 0)))
    def _():
        s1_ref[...] = s1_ref[...] + part


def _pass_c_body(y1_ref, a1_ref, c1_ref, out_ref):
    out_ref[0] = jnp.maximum(y1_ref[0] * a1_ref[...] + c1_ref[...], 0.0)


def kernel(xyz1, xyz2, feat1, feat2, W0, b0, g0, beta0, W1, b1, g1, beta1):
    xyz2t = jnp.transpose(xyz2, (0, 2, 1))
    w0a = W0[:, :C1]
    w0b = W0[:, C1:]
    ntot = jnp.float32(B * N1)

    idxg, wn, y0a, table = pl.pallas_call(
        _pass_a_body,
        grid=(B, N1 // NBLK),
        in_specs=[
            pl.BlockSpec((1, NBLK, 3), lambda b, i: (b, i, 0)),
            pl.BlockSpec((1, 3, N2), lambda b, i: (b, 0, 0)),
            pl.BlockSpec((1, C1, NBLK), lambda b, i: (b, 0, i)),
            pl.BlockSpec((1, C2, N2), lambda b, i: (b, 0, 0)),
            pl.BlockSpec((C2, C1), lambda b, i: (0, 0)),
            pl.BlockSpec((C2, C2), lambda b, i: (0, 0)),
            pl.BlockSpec((1, C2), lambda b, i: (0, 0)),
        ],
        out_specs=[
            pl.BlockSpec((1, NBLK, 3), lambda b, i: (b, i, 0)),
            pl.BlockSpec((1, NBLK, 48), lambda b, i: (b, i, 0)),
            pl.BlockSpec((1, NBLK, C2), lambda b, i: (b, i, 0)),
            pl.BlockSpec((1, N2, C2), lambda b, i: (b, 0, 0)),
        ],
        out_shape=[
            jax.ShapeDtypeStruct((B, N1, 3), jnp.int32),
            jax.ShapeDtypeStruct((B, N1, 48), jnp.float32),
            jax.ShapeDtypeStruct((B, N1, C2), jnp.float32),
            jax.ShapeDtypeStruct((B, N2, C2), jnp.float32),
        ],
    )(xyz1, xyz2t, feat1, feat2, w0a, w0b, b0.reshape(1, C2))

    mesh = plsc.VectorSubcoreMesh(core_axis_name="c", subcore_axis_name="s")
    interp = functools.partial(
        pl.kernel,
        mesh=mesh,
        out_type=jax.ShapeDtypeStruct((B * N1, C2), jnp.float32),
        scratch_types=[
            pltpu.VMEM((RPW * 3,), jnp.int32),
            pltpu.VMEM((2, CH * 3, 16), jnp.float32),
            pltpu.VMEM((2, CH * 3, C2), jnp.float32),
            pltpu.VMEM((2, CH, C2), jnp.float32),
            pltpu.SemaphoreType.DMA,
            pltpu.SemaphoreType.DMA,
            pltpu.SemaphoreType.DMA,
            pltpu.SemaphoreType.DMA,
        ],
    )(_sc_body)(
        table.reshape(B * N2, C2),
        idxg.reshape(B * N1 * 3),
        wn.reshape(B * N1 * 3, 16),
    )

    y0, s0 = pl.pallas_call(
        _pass_s_body,
        grid=(B, N1 // NBLK_S),
        in_specs=[
            pl.BlockSpec((1, NBLK_S, C2), lambda b, i: (b, i, 0)),
            pl.BlockSpec((1, NBLK_S, C2), lambda b, i: (b, i, 0)),
        ],
        out_specs=[
            pl.BlockSpec((1, NBLK_S, C2), lambda b, i: (b, i, 0)),
            pl.BlockSpec((2, C2), lambda b, i: (0, 0)),
        ],
        out_shape=[
            jax.ShapeDtypeStruct((B, N1, C2), jnp.float32),
            jax.ShapeDtypeStruct((2, C2), jnp.float32),
        ],
    )(interp.reshape(B, N1, C2), y0a)

    mean0 = s0[0] / ntot
    var0 = s0[1] / ntot - mean0 * mean0
    a0 = g0 / jnp.sqrt(var0 + 1e-5)
    c0 = beta0 - mean0 * a0

    y1, s1 = pl.pallas_call(
        _pass_b_body,
        grid=(B, N1 // NBLK_B),
        in_specs=[
            pl.BlockSpec((1, NBLK_B, C2), lambda b, i: (b, i, 0)),
            pl.BlockSpec((1, C2), lambda b, i: (0, 0)),
            pl.BlockSpec((1, C2), lambda b, i: (0, 0)),
            pl.BlockSpec((C1, C2), lambda b, i: (0, 0)),
            pl.BlockSpec((C1, 1), lambda b, i: (0, 0)),
        ],
        out_specs=[
            pl.BlockSpec((1, C1, NBLK_B), lambda b, i: (b, 0, i)),
            pl.BlockSpec((C1, 2), lambda b, i: (0, 0)),
        ],
        out_shape=[
            jax.ShapeDtypeStruct((B, C1, N1), jnp.float32),
            jax.ShapeDtypeStruct((C1, 2), jnp.float32),
        ],
    )(y0, a0.reshape(1, C2), c0.reshape(1, C2), W1,
      b1.reshape(C1, 1))

    mean1 = s1[:, 0] / ntot
    var1 = s1[:, 1] / ntot - mean1 * mean1
    a1 = g1 / jnp.sqrt(var1 + 1e-5)
    c1 = beta1 - mean1 * a1

    out = pl.pallas_call(
        _pass_c_body,
        grid=(B,),
        in_specs=[
            pl.BlockSpec((1, C1, N1), lambda b: (b, 0, 0)),
            pl.BlockSpec((C1, 1), lambda b: (0, 0)),
            pl.BlockSpec((C1, 1), lambda b: (0, 0)),
        ],
        out_specs=pl.BlockSpec((1, C1, N1), lambda b: (b, 0, 0)),
        out_shape=jax.ShapeDtypeStruct((B, C1, N1), jnp.float32),
    )(y1, a1.reshape(C1, 1), c1.reshape(C1, 1))

    return out


# speedup vs baseline: 1.0804x; 1.0804x over previous
import functools

import jax
import jax.numpy as jnp
from jax import lax
from jax.experimental import pallas as pl
from jax.experimental.pallas import tpu as pltpu
from jax.experimental.pallas import tpu_sc as plsc

B, N1, N2 = 8, 4096, 1024
C1, C2 = 128, 256
NBLK = 1024
NBLK_B = 2048
NBLK_S = 1024

NW = 32
RPW = (B * N1) // NW
CH = 16
NCH = RPW // CH


def _pass_a_body(xyz1_ref, xyz2t_ref, feat1_ref, feat2_ref, w0a_ref, w0b_ref,
                 b0_ref, idx_ref, wn_ref, y0a_ref, table_ref):
    b = pl.program_id(0)
    i = pl.program_id(1)

    @pl.when(i == 0)
    def _():
        table_ref[0] = jax.lax.dot_general(
            feat2_ref[0], w0b_ref[...], (((0,), (1,)), ((), ())),
            preferred_element_type=jnp.float32)

    x1 = xyz1_ref[0]
    x2 = xyz2t_ref[0]
    dx = x1[:, 0:1] - x2[0:1, :]
    dy = x1[:, 1:2] - x2[1:2, :]
    dz = x1[:, 2:3] - x2[2:3, :]
    d = (dx * dx + dy * dy) + dz * dz

    iota_i = jax.lax.broadcasted_iota(jnp.int32, (NBLK, N2), 1)
    db = jax.lax.bitcast_convert_type(d, jnp.int32)
    key = (db & jnp.int32(-1024)) | iota_i
    bigi = jnp.int32(0x7FFFFFFF)
    k1 = jnp.min(key, axis=1, keepdims=True)
    key2 = jnp.where(key == k1, bigi, key)
    k2 = jnp.min(key2, axis=1, keepdims=True)
    key3 = jnp.where(key2 == k2, bigi, key2)
    k3 = jnp.min(key3, axis=1, keepdims=True)
    i1 = k1 & 1023
    i2 = k2 & 1023
    i3 = k3 & 1023
    m1 = jax.lax.bitcast_convert_type(k1 & jnp.int32(-1024), jnp.float32)
    m2 = jax.lax.bitcast_convert_type(k2 & jnp.int32(-1024), jnp.float32)
    m3 = jax.lax.bitcast_convert_type(k3 & jnp.int32(-1024), jnp.float32)
    w1 = 1.0 / (m1 + 1e-8)
    w2 = 1.0 / (m2 + 1e-8)
    w3 = 1.0 / (m3 + 1e-8)
    wsum = w1 + w2 + w3

    idx_ref[0] = jnp.concatenate([i1, i2, i3], axis=1) + b * N2
    ones16 = jnp.ones((1, 16), jnp.float32)
    wn_ref[0] = jnp.concatenate(
        [(w1 / wsum) * ones16, (w2 / wsum) * ones16, (w3 / wsum) * ones16],
        axis=1)

    y0a_ref[0] = jax.lax.dot_general(
        feat1_ref[0], w0a_ref[...], (((0,), (1,)), ((), ())),
        preferred_element_type=jnp.float32) + b0_ref[...]


def _sc_body(table_hbm, idx_hbm, w_hbm, y0_hbm,
             idx_all, w_v, rows_v, out_v,
             gsem0, gsem1, osem0, osem1):
    wid = lax.axis_index("s") * 2 + lax.axis_index("c")
    base = wid * RPW
    CH3 = CH * 3
    gsems = (gsem0, gsem1)
    osems = (osem0, osem1)

    pltpu.sync_copy(idx_hbm.at[pl.ds(base * 3, RPW * 3)], idx_all)

    def start(t, s):
        r0 = base + t * CH
        pltpu.async_copy(table_hbm.at[idx_all.at[pl.ds(t * CH3, CH3)]],
                         rows_v.at[s], gsems[s])
        pltpu.async_copy(w_hbm.at[pl.ds(r0 * 3, CH3)], w_v.at[s], gsems[s])

    def wait_in(t, s):
        pltpu.make_async_copy(table_hbm.at[idx_all.at[pl.ds(t * CH3, CH3)]],
                              rows_v.at[s], gsems[s]).wait()
        pltpu.make_async_copy(w_hbm.at[pl.ds(0, CH3)], w_v.at[s],
                              gsems[s]).wait()

    def wait_out(os_):
        pltpu.make_async_copy(out_v.at[os_], y0_hbm.at[pl.ds(0, CH)],
                              osems[os_]).wait()

    def compute(t, s, os_):
        r0 = base + t * CH
        for q in range(CH):
            w0 = w_v[s, 3 * q, :]
            w1 = w_v[s, 3 * q + 1, :]
            w2 = w_v[s, 3 * q + 2, :]
            for c in range(C2 // 16):
                sl = pl.ds(c * 16, 16)
                acc = (w0 * rows_v[s, 3 * q, sl]
                       + w1 * rows_v[s, 3 * q + 1, sl]
                       + w2 * rows_v[s, 3 * q + 2, sl])
                out_v[os_, q, sl] = acc
        pltpu.async_copy(out_v.at[os_], y0_hbm.at[pl.ds(r0, CH)], osems[os_])

    start(0, 0)
    start(1, 1)

    def body(tt, carry):
        t0 = tt * 2
        for s in range(2):
            t = t0 + s
            wait_in(t, s)

            @pl.when(t >= 2)
            def _():
                wait_out(s)

            compute(t, s, s)

            @pl.when(t + 2 < NCH)
            def _():
                start(t + 2, s)

        return carry

    lax.fori_loop(0, NCH // 2, body, 0)
    wait_out(0)
    wait_out(1)


def _pass_s_body(interp_ref, y0a_ref, y0_ref, s0_ref):
    b = pl.program_id(0)
    i = pl.program_id(1)
    y0 = interp_ref[0] + y0a_ref[0]
    y0_ref[0] = y0
    part = jnp.concatenate(
        [jnp.sum(y0, axis=0, keepdims=True),
         jnp.sum(y0 * y0, axis=0, keepdims=True)], axis=0)

    @pl.when((b == 0) & (i == 0))
    def _():
        s0_ref[...] = part

    @pl.when(~((b == 0) & (i == 0)))
    def _():
        s0_ref[...] = s0_ref[...] + part


def _pass_b_body(y0_ref, a0_ref, c0_ref, w1_ref, b1_ref, y1_ref, s1_ref):
    b = pl.program_id(0)
    i = pl.program_id(1)
    h0 = jnp.maximum(y0_ref[0] * a0_ref[...] + c0_ref[...], 0.0)
    y1 = jax.lax.dot_general(
        w1_ref[...], h0, (((1,), (1,)), ((), ())),
        preferred_element_type=jnp.float32) + b1_ref[...]
    y1_ref[0] = y1

    part = jnp.concatenate(
        [jnp.sum(y1, axis=1, keepdims=True),
         jnp.sum(y1 * y1, axis=1, keepdims=True)], axis=1)

    @pl.when((b == 0) & (i == 0))
    def _():
        s1_ref[...] = part

    @pl.when(~((b == 0) & (i == 0)))
    def _():
        s1_ref[...] = s1_ref[...] + part


def _pass_c_body(y1_ref, a1_ref, c1_ref, out_ref):
    out_ref[0] = jnp.maximum(y1_ref[0] * a1_ref[...] + c1_ref[...], 0.0)


def kernel(xyz1, xyz2, feat1, feat2, W0, b0, g0, beta0, W1, b1, g1, beta1):
    xyz2t = jnp.transpose(xyz2, (0, 2, 1))
    w0a = W0[:, :C1]
    w0b = W0[:, C1:]
    ntot = jnp.float32(B * N1)

    idxg, wn, y0a, table = pl.pallas_call(
        _pass_a_body,
        grid=(B, N1 // NBLK),
        in_specs=[
            pl.BlockSpec((1, NBLK, 3), lambda b, i: (b, i, 0)),
            pl.BlockSpec((1, 3, N2), lambda b, i: (b, 0, 0)),
            pl.BlockSpec((1, C1, NBLK), lambda b, i: (b, 0, i)),
            pl.BlockSpec((1, C2, N2), lambda b, i: (b, 0, 0)),
            pl.BlockSpec((C2, C1), lambda b, i: (0, 0)),
            pl.BlockSpec((C2, C2), lambda b, i: (0, 0)),
            pl.BlockSpec((1, C2), lambda b, i: (0, 0)),
        ],
        out_specs=[
            pl.BlockSpec((1, NBLK, 3), lambda b, i: (b, i, 0)),
            pl.BlockSpec((1, NBLK, 48), lambda b, i: (b, i, 0)),
            pl.BlockSpec((1, NBLK, C2), lambda b, i: (b, i, 0)),
            pl.BlockSpec((1, N2, C2), lambda b, i: (b, 0, 0)),
        ],
        out_shape=[
            jax.ShapeDtypeStruct((B, N1, 3), jnp.int32),
            jax.ShapeDtypeStruct((B, N1, 48), jnp.float32),
            jax.ShapeDtypeStruct((B, N1, C2), jnp.float32),
            jax.ShapeDtypeStruct((B, N2, C2), jnp.float32),
        ],
    )(xyz1, xyz2t, feat1, feat2, w0a, w0b, b0.reshape(1, C2))

    mesh = plsc.VectorSubcoreMesh(core_axis_name="c", subcore_axis_name="s")
    interp = functools.partial(
        pl.kernel,
        mesh=mesh,
        out_type=jax.ShapeDtypeStruct((B * N1, C2), jnp.float32),
        scratch_types=[
            pltpu.VMEM((RPW * 3,), jnp.int32),
            pltpu.VMEM((2, CH * 3, 16), jnp.float32),
            pltpu.VMEM((2, CH * 3, C2), jnp.float32),
            pltpu.VMEM((2, CH, C2), jnp.float32),
            pltpu.SemaphoreType.DMA,
            pltpu.SemaphoreType.DMA,
            pltpu.SemaphoreType.DMA,
            pltpu.SemaphoreType.DMA,
        ],
    )(_sc_body)(
        table.reshape(B * N2, C2),
        idxg.reshape(B * N1 * 3),
        wn.reshape(B * N1 * 3, 16),
    )

    y0, s0 = pl.pallas_call(
        _pass_s_body,
        grid=(B, N1 // NBLK_S),
        in_specs=[
            pl.BlockSpec((1, NBLK_S, C2), lambda b, i: (b, i, 0)),
            pl.BlockSpec((1, NBLK_S, C2), lambda b, i: (b, i, 0)),
        ],
        out_specs=[
            pl.BlockSpec((1, NBLK_S, C2), lambda b, i: (b, i, 0)),
            pl.BlockSpec((2, C2), lambda b, i: (0, 0)),
        ],
        out_shape=[
            jax.ShapeDtypeStruct((B, N1, C2), jnp.float32),
            jax.ShapeDtypeStruct((2, C2), jnp.float32),
        ],
    )(interp.reshape(B, N1, C2), y0a)

    mean0 = s0[0] / ntot
    var0 = s0[1] / ntot - mean0 * mean0
    a0 = g0 / jnp.sqrt(var0 + 1e-5)
    c0 = beta0 - mean0 * a0

    y1, s1 = pl.pallas_call(
        _pass_b_body,
        grid=(B, N1 // NBLK_B),
        in_specs=[
            pl.BlockSpec((1, NBLK_B, C2), lambda b, i: (b, i, 0)),
            pl.BlockSpec((1, C2), lambda b, i: (0, 0)),
            pl.BlockSpec((1, C2), lambda b, i: (0, 0)),
            pl.BlockSpec((C1, C2), lambda b, i: (0, 0)),
            pl.BlockSpec((C1, 1), lambda b, i: (0, 0)),
        ],
        out_specs=[
            pl.BlockSpec((1, C1, NBLK_B), lambda b, i: (b, 0, i)),
            pl.BlockSpec((C1, 2), lambda b, i: (0, 0)),
        ],
        out_shape=[
            jax.ShapeDtypeStruct((B, C1, N1), jnp.float32),
            jax.ShapeDtypeStruct((C1, 2), jnp.float32),
        ],
    )(y0, a0.reshape(1, C2), c0.reshape(1, C2), W1,
      b1.reshape(C1, 1))

    mean1 = s1[:, 0] / ntot
    var1 = s1[:, 1] / ntot - mean1 * mean1
    a1 = g1 / jnp.sqrt(var1 + 1e-5)
    c1 = beta1 - mean1 * a1

    out = pl.pallas_call(
        _pass_c_body,
        grid=(B,),
        in_specs=[
            pl.BlockSpec((1, C1, N1), lambda b: (b, 0, 0)),
            pl.BlockSpec((C1, 1), lambda b: (0, 0)),
            pl.BlockSpec((C1, 1), lambda b: (0, 0)),
        ],
        out_specs=pl.BlockSpec((1, C1, N1), lambda b: (b, 0, 0)),
        out_shape=jax.ShapeDtypeStruct((B, C1, N1), jnp.float32),
    )(y1, a1.reshape(C1, 1), c1.reshape(C1, 1))

    return out
